# Initial kernel scaffold; baseline (speedup 1.0000x reference)
import jax
import jax.numpy as jnp
from jax.experimental import pallas as pl


def kernel(x, edge_attr, senders, receivers, n_atoms, eW1, eb1, eW2, eb2, eW3, eb3, eg, ebt, nW1, nb1, nW2, nb2, nW3, nb3, ng, nbt):
    raise NotImplementedError("write your pallas kernel here")


# speedup vs baseline: 3.2463x; 3.2463x over previous
import functools

import jax
import jax.numpy as jnp
from jax import lax
from jax.experimental import pallas as pl
from jax.experimental.pallas import tpu as pltpu
from jax.experimental.pallas import tpu_sc as plsc

_N = 10000
_K = 16
_D = 128
_H = 256
_E = _N * _K


_CH = 128
_NCHUNK = _E // _CH
_NC = 2
_NS = 16
_NW = _NC * _NS


def _sc_gather_kernel(x_hbm, recv_hbm, send_hbm, rec_out, snd_out,
                      ridx, rrows, sidx, srows, rsem, ssem):
    wid = lax.axis_index("s") * _NC + lax.axis_index("c")
    nt = (_NCHUNK - wid + _NW - 1) // _NW

    def body(t, carry):
        base = (wid + t * _NW) * _CH
        pltpu.sync_copy(recv_hbm.at[pl.ds(base, _CH)], ridx)
        pltpu.sync_copy(send_hbm.at[pl.ds(base, _CH)], sidx)
        r1 = pltpu.async_copy(x_hbm.at[ridx], rrows, rsem)
        r2 = pltpu.async_copy(x_hbm.at[sidx], srows, ssem)
        r1.wait()
        r2.wait()
        pltpu.sync_copy(rrows, rec_out.at[pl.ds(base, _CH)])
        pltpu.sync_copy(srows, snd_out.at[pl.ds(base, _CH)])
        return carry

    lax.fori_loop(0, nt, body, 0)


@functools.cache
def _sc_gather():
    return pl.kernel(
        _sc_gather_kernel,
        mesh=plsc.VectorSubcoreMesh(core_axis_name="c", subcore_axis_name="s"),
        out_type=(
            jax.ShapeDtypeStruct((_E, _D), jnp.float32),
            jax.ShapeDtypeStruct((_E, _D), jnp.float32),
        ),
        scratch_types=[
            pltpu.VMEM((_CH,), jnp.int32),
            pltpu.VMEM((_CH, _D), jnp.float32),
            pltpu.VMEM((_CH,), jnp.int32),
            pltpu.VMEM((_CH, _D), jnp.float32),
            pltpu.SemaphoreType.DMA,
            pltpu.SemaphoreType.DMA,
        ],
    )


_NB = 80
_BE = _NB * _K
_GRID = _N // _NB


def _ln(h, g, bt):
    mu = jnp.mean(h, axis=-1, keepdims=True)
    var = jnp.mean((h - mu) * (h - mu), axis=-1, keepdims=True)
    return g * ((h - mu) * lax.rsqrt(var + 1e-5)) + bt


def _tc_kernel(rec, snd, ea,
               w1r, w1s, w1e, b1, w2, b2, w3, b3, g, bt,
               m_out, s_out):
    f32 = jnp.float32
    h = jnp.dot(rec[...], w1r[...], preferred_element_type=f32)
    h = h + jnp.dot(snd[...], w1s[...], preferred_element_type=f32)
    h = h + jnp.dot(ea[...], w1e[...], preferred_element_type=f32)
    h = jax.nn.relu(h + b1[...])
    h = jax.nn.relu(jnp.dot(h, w2[...], preferred_element_type=f32) + b2[...])
    m = jnp.dot(h, w3[...], preferred_element_type=f32) + b3[...]
    mln = _ln(m, g[...], bt[...])
    m_out[...] = mln
    grp = (lax.broadcasted_iota(jnp.int32, (_D, 8), 0) // 16
           == lax.broadcasted_iota(jnp.int32, (_D, 8), 1))
    s_out[...] = jnp.dot(mln, grp.astype(f32), preferred_element_type=f32)


def _node_kernel(xin, nw1, nb1, nw2, nb2, nw3, nb3, ng, nbt, x_out):
    f32 = jnp.float32
    h = jax.nn.relu(jnp.dot(xin[...], nw1[...], preferred_element_type=f32) + nb1[...])
    h = jax.nn.relu(jnp.dot(h, nw2[...], preferred_element_type=f32) + nb2[...])
    y = jnp.dot(h, nw3[...], preferred_element_type=f32) + nb3[...]
    x_out[...] = _ln(y, ng[...], nbt[...])


def _const(shape):
    return pl.BlockSpec(shape, lambda i: tuple(0 for _ in shape))


_NNB = 1000


def _tc_call(rec, snd, ea, ws, interpret=False):
    (w1r, w1s, w1e, b1, w2, b2, w3, b3, g, bt,
     nw1, nb1, nw2, nb2, nw3, nb3, ng, nbt) = ws
    m, s = pl.pallas_call(
        _tc_kernel,
        grid=(_GRID,),
        in_specs=[
            pl.BlockSpec((_BE, _D), lambda i: (i, 0)),
            pl.BlockSpec((_BE, _D), lambda i: (i, 0)),
            pl.BlockSpec((_BE, _D), lambda i: (i, 0)),
            _const((_D, _H)), _const((_D, _H)), _const((_D, _H)),
            _const((1, _H)),
            _const((_H, _H)), _const((1, _H)),
            _const((_H, _D)), _const((1, _D)),
            _const((1, _D)), _const((1, _D)),
        ],
        out_specs=[
            pl.BlockSpec((_BE, _D), lambda i: (i, 0)),
            pl.BlockSpec((_BE, 8), lambda i: (i, 0)),
        ],
        out_shape=[
            jax.ShapeDtypeStruct((_E, _D), jnp.float32),
            jax.ShapeDtypeStruct((_E, 8), jnp.float32),
        ],
        interpret=interpret,
    )(rec, snd, ea, w1r, w1s, w1e, b1, w2, b2, w3, b3, g, bt)
    xin = s.reshape(_N, _D)
    x_out = pl.pallas_call(
        _node_kernel,
        grid=(_N // _NNB,),
        in_specs=[
            pl.BlockSpec((_NNB, _D), lambda i: (i, 0)),
            _const((_D, _H)), _const((1, _H)),
            _const((_H, _H)), _const((1, _H)),
            _const((_H, _D)), _const((1, _D)),
            _const((1, _D)), _const((1, _D)),
        ],
        out_specs=pl.BlockSpec((_NNB, _D), lambda i: (i, 0)),
        out_shape=jax.ShapeDtypeStruct((_N, _D), jnp.float32),
        interpret=interpret,
    )(xin, nw1, nb1, nw2, nb2, nw3, nb3, ng, nbt)
    return m, x_out


def kernel(x, edge_attr, senders, receivers, n_atoms,
           eW1, eb1, eW2, eb2, eW3, eb3, eg, ebt,
           nW1, nb1, nW2, nb2, nW3, nb3, ng, nbt):
    rec_feat, snd_feat = _sc_gather()(x, receivers, senders)
    ws = (eW1[:_D], eW1[_D:2 * _D], eW1[2 * _D:],
          eb1.reshape(1, _H), eW2, eb2.reshape(1, _H),
          eW3, eb3.reshape(1, _D), eg.reshape(1, _D), ebt.reshape(1, _D),
          nW1, nb1.reshape(1, _H), nW2, nb2.reshape(1, _H),
          nW3, nb3.reshape(1, _D), ng.reshape(1, _D), nbt.reshape(1, _D))
    m, x_out = _tc_call(rec_feat, snd_feat, edge_attr, ws)
    return (x_out, m)
